# baseline (speedup 1.0000x reference)
import functools

import jax
import jax.numpy as jnp
from jax import lax
from jax.experimental import pallas as pl
from jax.experimental.pallas import tpu as pltpu
from jax.experimental.pallas import tpu_sc as plsc

M = 50000
R = 100000
D = 128
NC = 2
NS = 16
NW = NC * NS
TR = R // 8
TR_BASE = TR // NW
TR_EXTRA = TR % NW
EDGES = [8 * (w * TR_BASE + min(w, TR_EXTRA)) for w in range(NW + 1)]
ZROWS = 256
NZ = -(-max(EDGES[w + 1] - EDGES[w] for w in range(NW)) // ZROWS)
C = 128
BIG = 1 << 30


NB = 4


def _body(h_hbm, idx_hbm, src_hbm, bounds_hbm, z_hbm, out_hbm,
          bv, zbuf, ib, sb, hb0, hb1, hb2, hb3, zsem, isem, gsem, ssem):
    hbs = [hb0, hb1, hb2, hb3]
    cid = lax.axis_index("c")
    sid = lax.axis_index("s")
    wid = sid * NC + cid
    r0 = 8 * (wid * TR_BASE + jnp.minimum(wid, TR_EXTRA))
    r1 = 8 * ((wid + 1) * TR_BASE + jnp.minimum(wid + 1, TR_EXTRA))

    pltpu.sync_copy(z_hbm, zbuf)
    zcopies = [
        pltpu.async_copy(
            zbuf,
            out_hbm.at[pl.ds(jnp.minimum(r0 + k * ZROWS, r1 - ZROWS), ZROWS)],
            zsem)
        for k in range(NZ)
    ]

    pltpu.sync_copy(bounds_hbm, bv)
    vregs = [bv[pl.ds(k * 16, 16)] for k in range(3)]

    def bval(i):
        s = jnp.int32(0)
        for k in range(3):
            for l in range(16):
                pos = k * 16 + l
                if pos <= NW:
                    s = jnp.where(i == pos, vregs[k][l], s)
        return s

    b0 = bval(wid)
    b1 = bval(wid + 1)

    a0 = b0 & jnp.int32(-8)
    nc = jnp.maximum((b1 - a0 + (C - 1)) // C, 0)
    niter = (nc + NB - 1) // NB

    def cstart(cid):
        return pl.multiple_of(jnp.clip(a0 + cid * C, 0, M - C), 8)

    def group(g, carry):
        for s in range(NB):
            cid = g * NB + s

            @pl.when(jnp.logical_and(g > 0, (g - 1) * NB + s < nc))
            def _():
                pltpu.make_async_copy(hbs[s], out_hbm.at[ib.at[s]], ssem).wait()

            @pl.when(cid < nc)
            def _():
                st = cstart(cid)
                pltpu.async_copy(idx_hbm.at[pl.ds(st, C)], ib.at[s], isem)
                pltpu.async_copy(src_hbm.at[pl.ds(st, C)], sb.at[s], isem)

        for s in range(NB):
            cid = g * NB + s

            @pl.when(cid < nc)
            def _():
                pltpu.make_async_copy(
                    idx_hbm.at[pl.ds(0, C)], ib.at[s], isem).wait()
                pltpu.make_async_copy(
                    src_hbm.at[pl.ds(0, C)], sb.at[s], isem).wait()
                pltpu.async_copy(h_hbm.at[sb.at[s]], hbs[s], gsem)

        @pl.when(g == 0)
        def _():
            for cp in zcopies:
                cp.wait()

        for s in range(NB):
            cid = g * NB + s

            @pl.when(cid < nc)
            def _():
                pltpu.make_async_copy(h_hbm.at[sb.at[s]], hbs[s], gsem).wait()
                pltpu.async_copy(hbs[s], out_hbm.at[ib.at[s]], ssem)

        return carry

    lax.fori_loop(0, niter, group, jnp.int32(0))

    for s in range(NB):
        @pl.when(jnp.logical_and(nc > 0, (niter - 1) * NB + s < nc))
        def _():
            pltpu.make_async_copy(hbs[s], out_hbm.at[ib.at[s]], ssem).wait()

    @pl.when(niter == 0)
    def _():
        for cp in zcopies:
            cp.wait()


def _sc_unpool(h, idx32, src, bounds_pad, zrows):
    mesh = plsc.VectorSubcoreMesh(core_axis_name="c", subcore_axis_name="s")
    return pl.kernel(
        _body,
        out_type=jax.ShapeDtypeStruct((R, D), jnp.float32),
        mesh=mesh,
        scratch_types=[
            pltpu.VMEM((48,), jnp.int32),
            pltpu.VMEM((ZROWS, D), jnp.float32),
            pltpu.VMEM((NB, C), jnp.int32),
            pltpu.VMEM((NB, C), jnp.int32),
            pltpu.VMEM((C, D), jnp.float32),
            pltpu.VMEM((C, D), jnp.float32),
            pltpu.VMEM((C, D), jnp.float32),
            pltpu.VMEM((C, D), jnp.float32),
            pltpu.SemaphoreType.DMA,
            pltpu.SemaphoreType.DMA,
            pltpu.SemaphoreType.DMA,
            pltpu.SemaphoreType.DMA,
        ],
    )(h, idx32, src, bounds_pad, zrows)


def kernel(h, pre_node_num, idx):
    del pre_node_num
    idx32 = idx.astype(jnp.int32)
    j = jnp.arange(M, dtype=jnp.int32)
    nxt = jnp.concatenate([idx32[1:], jnp.full((1,), -1, jnp.int32)])
    src = jnp.where(idx32 != nxt, j, jnp.int32(M))
    d = 1
    while d < M:
        src = jnp.minimum(src, jnp.concatenate(
            [src[d:], jnp.full((d,), BIG, jnp.int32)]))
        d *= 2
    edges = jnp.asarray(EDGES, dtype=jnp.int32)
    bounds = jnp.sum(idx32[None, :] < edges[:, None], axis=1).astype(jnp.int32)
    bounds_pad = jnp.zeros((48,), jnp.int32).at[: NW + 1].set(bounds)
    zrows = jnp.zeros((ZROWS, D), jnp.float32)
    return _sc_unpool(h, idx32, src, bounds_pad, zrows)


# speedup vs baseline: 1.0302x; 1.0302x over previous
import functools

import jax
import jax.numpy as jnp
from jax import lax
from jax.experimental import pallas as pl
from jax.experimental.pallas import tpu as pltpu
from jax.experimental.pallas import tpu_sc as plsc

M = 50000
R = 100000
D = 128
NC = 2
NS = 16
NW = NC * NS
TR = R // 8
TR_BASE = TR // NW
TR_EXTRA = TR % NW
EDGES = [8 * (w * TR_BASE + min(w, TR_EXTRA)) for w in range(NW + 1)]
ZROWS = 128
NZ = -(-max(EDGES[w + 1] - EDGES[w] for w in range(NW)) // ZROWS)
C = 128
BIG = 1 << 30


NB = 6


def _body(h_hbm, idx_hbm, src_hbm, bounds_hbm, z_hbm, out_hbm,
          bv, zbuf, ib, sb, hb0, hb1, hb2, hb3, hb4, hb5,
          zsem, isem, gsem, ssem):
    hbs = [hb0, hb1, hb2, hb3, hb4, hb5]
    cid = lax.axis_index("c")
    sid = lax.axis_index("s")
    wid = sid * NC + cid
    r0 = 8 * (wid * TR_BASE + jnp.minimum(wid, TR_EXTRA))
    r1 = 8 * ((wid + 1) * TR_BASE + jnp.minimum(wid + 1, TR_EXTRA))

    pltpu.sync_copy(z_hbm, zbuf)
    zcopies = [
        pltpu.async_copy(
            zbuf,
            out_hbm.at[pl.ds(jnp.minimum(r0 + k * ZROWS, r1 - ZROWS), ZROWS)],
            zsem)
        for k in range(NZ)
    ]

    pltpu.sync_copy(bounds_hbm, bv)
    vregs = [bv[pl.ds(k * 16, 16)] for k in range(3)]

    def bval(i):
        s = jnp.int32(0)
        for k in range(3):
            for l in range(16):
                pos = k * 16 + l
                if pos <= NW:
                    s = jnp.where(i == pos, vregs[k][l], s)
        return s

    b0 = bval(wid)
    b1 = bval(wid + 1)

    a0 = b0 & jnp.int32(-8)
    nc = jnp.maximum((b1 - a0 + (C - 1)) // C, 0)
    niter = (nc + NB - 1) // NB

    def cstart(cid):
        return pl.multiple_of(jnp.clip(a0 + cid * C, 0, M - C), 8)

    def group(g, carry):
        for s in range(NB):
            cid = g * NB + s

            @pl.when(jnp.logical_and(g > 0, (g - 1) * NB + s < nc))
            def _():
                pltpu.make_async_copy(hbs[s], out_hbm.at[ib.at[s]], ssem).wait()

            @pl.when(cid < nc)
            def _():
                st = cstart(cid)
                pltpu.async_copy(idx_hbm.at[pl.ds(st, C)], ib.at[s], isem)
                pltpu.async_copy(src_hbm.at[pl.ds(st, C)], sb.at[s], isem)

        for s in range(NB):
            cid = g * NB + s

            @pl.when(cid < nc)
            def _():
                pltpu.make_async_copy(
                    idx_hbm.at[pl.ds(0, C)], ib.at[s], isem).wait()
                pltpu.make_async_copy(
                    src_hbm.at[pl.ds(0, C)], sb.at[s], isem).wait()
                pltpu.async_copy(h_hbm.at[sb.at[s]], hbs[s], gsem)

        @pl.when(g == 0)
        def _():
            for cp in zcopies:
                cp.wait()

        for s in range(NB):
            cid = g * NB + s

            @pl.when(cid < nc)
            def _():
                pltpu.make_async_copy(h_hbm.at[sb.at[s]], hbs[s], gsem).wait()
                pltpu.async_copy(hbs[s], out_hbm.at[ib.at[s]], ssem)

        return carry

    lax.fori_loop(0, niter, group, jnp.int32(0))

    for s in range(NB):
        @pl.when(jnp.logical_and(nc > 0, (niter - 1) * NB + s < nc))
        def _():
            pltpu.make_async_copy(hbs[s], out_hbm.at[ib.at[s]], ssem).wait()

    @pl.when(niter == 0)
    def _():
        for cp in zcopies:
            cp.wait()


def _sc_unpool(h, idx32, src, bounds_pad, zrows):
    mesh = plsc.VectorSubcoreMesh(core_axis_name="c", subcore_axis_name="s")
    return pl.kernel(
        _body,
        out_type=jax.ShapeDtypeStruct((R, D), jnp.float32),
        mesh=mesh,
        scratch_types=[
            pltpu.VMEM((48,), jnp.int32),
            pltpu.VMEM((ZROWS, D), jnp.float32),
            pltpu.VMEM((NB, C), jnp.int32),
            pltpu.VMEM((NB, C), jnp.int32),
            pltpu.VMEM((C, D), jnp.float32),
            pltpu.VMEM((C, D), jnp.float32),
            pltpu.VMEM((C, D), jnp.float32),
            pltpu.VMEM((C, D), jnp.float32),
            pltpu.VMEM((C, D), jnp.float32),
            pltpu.VMEM((C, D), jnp.float32),
            pltpu.SemaphoreType.DMA,
            pltpu.SemaphoreType.DMA,
            pltpu.SemaphoreType.DMA,
            pltpu.SemaphoreType.DMA,
        ],
    )(h, idx32, src, bounds_pad, zrows)


def kernel(h, pre_node_num, idx):
    del pre_node_num
    idx32 = idx.astype(jnp.int32)
    j = jnp.arange(M, dtype=jnp.int32)
    nxt = jnp.concatenate([idx32[1:], jnp.full((1,), -1, jnp.int32)])
    src = jnp.where(idx32 != nxt, j, jnp.int32(M))
    d = 1
    while d < M:
        src = jnp.minimum(src, jnp.concatenate(
            [src[d:], jnp.full((d,), BIG, jnp.int32)]))
        d *= 2
    edges = jnp.asarray(EDGES, dtype=jnp.int32)
    bounds = jnp.sum(idx32[None, :] < edges[:, None], axis=1).astype(jnp.int32)
    bounds_pad = jnp.zeros((48,), jnp.int32).at[: NW + 1].set(bounds)
    zrows = jnp.zeros((ZROWS, D), jnp.float32)
    return _sc_unpool(h, idx32, src, bounds_pad, zrows)
